# baseline (speedup 1.0000x reference)
import functools

import jax
import jax.numpy as jnp
from jax import lax
from jax.experimental import pallas as pl
from jax.experimental.pallas import tpu as pltpu
from jax.experimental.pallas import tpu_sc as plsc

_NUM_CORES = 2
_NUM_SUBCORES = 16
_NUM_WORKERS = _NUM_CORES * _NUM_SUBCORES

_LANES = 16
_UNROLL = 16


def _build_gather(D, V, B, dtype):
    mesh = plsc.VectorSubcoreMesh(core_axis_name="c", subcore_axis_name="s")
    n_full_rounds = D // _NUM_WORKERS
    rem_dims = D - n_full_rounds * _NUM_WORKERS

    blk_sz = 2048
    n_blocks = B // blk_sz

    @functools.partial(
        pl.kernel,
        out_type=jax.ShapeDtypeStruct((D, B), dtype),
        mesh=mesh,
        compiler_params=pltpu.CompilerParams(needs_layout_passes=False),
        scratch_types=[
            pltpu.VMEM((V,), dtype),
            pltpu.VMEM((B,), jnp.int32),
            pltpu.VMEM((blk_sz,), dtype),
            pltpu.VMEM((blk_sz,), dtype),
            pltpu.SemaphoreType.DMA,
            pltpu.SemaphoreType.DMA,
        ],
    )
    def gather(
        tableT_hbm, idx_hbm, outT_hbm, row_v, idx_v, out_a, out_b, sem_o, sem_r
    ):
        wid = lax.axis_index("s") * _NUM_CORES + lax.axis_index("c")
        pltpu.async_copy(idx_hbm, idx_v, sem_r)
        pltpu.async_copy(tableT_hbm.at[wid], row_v, sem_r)
        pltpu.make_async_copy(idx_hbm, idx_v, sem_r).wait()
        pltpu.make_async_copy(tableT_hbm.at[wid], row_v, sem_r).wait()
        for k in range(n_full_rounds):
            d = wid + _NUM_WORKERS * k
            if True:
                for blk in range(n_blocks):
                    buf = out_a if blk % 2 == 0 else out_b
                    if blk >= 2:
                        pltpu.make_async_copy(
                            buf,
                            outT_hbm.at[d, pl.ds((blk - 2) * blk_sz, blk_sz)],
                            sem_o,
                        ).wait()

                    @plsc.parallel_loop(
                        0, blk_sz, step=_LANES, unroll=_UNROLL
                    )
                    def sweep(o, blk=blk, buf=buf):
                        ids = idx_v[pl.ds(blk * blk_sz + o, _LANES)]
                        buf[pl.ds(o, _LANES)] = plsc.load_gather(
                            row_v, [ids]
                        )
                    pltpu.async_copy(
                        buf,
                        outT_hbm.at[d, pl.ds(blk * blk_sz, blk_sz)],
                        sem_o,
                    )
                if k + 1 < n_full_rounds:
                    pltpu.async_copy(
                        tableT_hbm.at[wid + _NUM_WORKERS * (k + 1)],
                        row_v,
                        sem_r,
                    )
                elif rem_dims:
                    @pl.when(wid < rem_dims)
                    def _prefetch_rem():
                        pltpu.async_copy(
                            tableT_hbm.at[wid + _NUM_WORKERS * k + _NUM_WORKERS],
                            row_v,
                            sem_r,
                        )

                for blk in (n_blocks - 2, n_blocks - 1):
                    pltpu.make_async_copy(
                        out_a if blk % 2 == 0 else out_b,
                        outT_hbm.at[d, pl.ds(blk * blk_sz, blk_sz)],
                        sem_o,
                    ).wait()
                if k + 1 < n_full_rounds:
                    pltpu.make_async_copy(
                        tableT_hbm.at[wid + _NUM_WORKERS * (k + 1)],
                        row_v,
                        sem_r,
                    ).wait()

        if rem_dims:
            @pl.when(wid < rem_dims)
            def _rem_round():
                d = wid + _NUM_WORKERS * n_full_rounds
                pltpu.make_async_copy(tableT_hbm.at[d], row_v, sem_r).wait()
                for blk in range(n_blocks):
                    buf = out_a if blk % 2 == 0 else out_b
                    if blk >= 2:
                        pltpu.make_async_copy(
                            buf,
                            outT_hbm.at[d, pl.ds((blk - 2) * blk_sz, blk_sz)],
                            sem_o,
                        ).wait()

                    @plsc.parallel_loop(
                        0, blk_sz, step=_LANES, unroll=_UNROLL
                    )
                    def sweep_rem(o, blk=blk, buf=buf):
                        ids = idx_v[pl.ds(blk * blk_sz + o, _LANES)]
                        buf[pl.ds(o, _LANES)] = plsc.load_gather(
                            row_v, [ids]
                        )
                    pltpu.async_copy(
                        buf,
                        outT_hbm.at[d, pl.ds(blk * blk_sz, blk_sz)],
                        sem_o,
                    )
                for blk in (n_blocks - 2, n_blocks - 1):
                    pltpu.make_async_copy(
                        out_a if blk % 2 == 0 else out_b,
                        outT_hbm.at[d, pl.ds(blk * blk_sz, blk_sz)],
                        sem_o,
                    ).wait()

    return gather


def kernel(embeddings, indices):
    (B,) = indices.shape
    V, D = embeddings.shape
    gather = _build_gather(D, V, B, embeddings.dtype)
    outT = gather(embeddings.T, indices.astype(jnp.int32))
    return outT.T


# speedup vs baseline: 1.0603x; 1.0603x over previous
import functools

import jax
import jax.numpy as jnp
from jax import lax
from jax.experimental import pallas as pl
from jax.experimental.pallas import tpu as pltpu
from jax.experimental.pallas import tpu_sc as plsc

_NUM_CORES = 2
_NUM_SUBCORES = 16
_NUM_WORKERS = _NUM_CORES * _NUM_SUBCORES

_LANES = 16
_UNROLL = 16


def _build_gather(D, V, B, dtype):
    mesh = plsc.VectorSubcoreMesh(core_axis_name="c", subcore_axis_name="s")
    n_full_rounds = D // _NUM_WORKERS
    rem_dims = D - n_full_rounds * _NUM_WORKERS

    blk_sz = 4096
    n_blocks = B // blk_sz

    @functools.partial(
        pl.kernel,
        out_type=jax.ShapeDtypeStruct((D, B), dtype),
        mesh=mesh,
        compiler_params=pltpu.CompilerParams(needs_layout_passes=False),
        scratch_types=[
            pltpu.VMEM((V,), dtype),
            pltpu.VMEM((B,), jnp.int32),
            pltpu.VMEM((blk_sz,), dtype),
            pltpu.VMEM((blk_sz,), dtype),
            pltpu.SemaphoreType.DMA,
            pltpu.SemaphoreType.DMA,
        ],
    )
    def gather(
        tableT_hbm, idx_hbm, outT_hbm, row_v, idx_v, out_a, out_b, sem_o, sem_r
    ):
        wid = lax.axis_index("s") * _NUM_CORES + lax.axis_index("c")
        pltpu.async_copy(idx_hbm, idx_v, sem_r)
        pltpu.async_copy(tableT_hbm.at[wid], row_v, sem_r)
        pltpu.make_async_copy(idx_hbm, idx_v, sem_r).wait()
        pltpu.make_async_copy(tableT_hbm.at[wid], row_v, sem_r).wait()
        for k in range(n_full_rounds):
            d = wid + _NUM_WORKERS * k
            if True:
                for blk in range(n_blocks):
                    buf = out_a if blk % 2 == 0 else out_b
                    if blk >= 2:
                        pltpu.make_async_copy(
                            buf,
                            outT_hbm.at[d, pl.ds((blk - 2) * blk_sz, blk_sz)],
                            sem_o,
                        ).wait()

                    @plsc.parallel_loop(
                        0, blk_sz, step=_LANES, unroll=_UNROLL
                    )
                    def sweep(o, blk=blk, buf=buf):
                        ids = idx_v[pl.ds(blk * blk_sz + o, _LANES)]
                        buf[pl.ds(o, _LANES)] = plsc.load_gather(
                            row_v, [ids]
                        )
                    pltpu.async_copy(
                        buf,
                        outT_hbm.at[d, pl.ds(blk * blk_sz, blk_sz)],
                        sem_o,
                    )
                if k + 1 < n_full_rounds:
                    pltpu.async_copy(
                        tableT_hbm.at[wid + _NUM_WORKERS * (k + 1)],
                        row_v,
                        sem_r,
                    )
                elif rem_dims:
                    @pl.when(wid < rem_dims)
                    def _prefetch_rem():
                        pltpu.async_copy(
                            tableT_hbm.at[wid + _NUM_WORKERS * k + _NUM_WORKERS],
                            row_v,
                            sem_r,
                        )

                for blk in (n_blocks - 2, n_blocks - 1):
                    pltpu.make_async_copy(
                        out_a if blk % 2 == 0 else out_b,
                        outT_hbm.at[d, pl.ds(blk * blk_sz, blk_sz)],
                        sem_o,
                    ).wait()
                if k + 1 < n_full_rounds:
                    pltpu.make_async_copy(
                        tableT_hbm.at[wid + _NUM_WORKERS * (k + 1)],
                        row_v,
                        sem_r,
                    ).wait()

        if rem_dims:
            @pl.when(wid < rem_dims)
            def _rem_round():
                d = wid + _NUM_WORKERS * n_full_rounds
                pltpu.make_async_copy(tableT_hbm.at[d], row_v, sem_r).wait()
                for blk in range(n_blocks):
                    buf = out_a if blk % 2 == 0 else out_b
                    if blk >= 2:
                        pltpu.make_async_copy(
                            buf,
                            outT_hbm.at[d, pl.ds((blk - 2) * blk_sz, blk_sz)],
                            sem_o,
                        ).wait()

                    @plsc.parallel_loop(
                        0, blk_sz, step=_LANES, unroll=_UNROLL
                    )
                    def sweep_rem(o, blk=blk, buf=buf):
                        ids = idx_v[pl.ds(blk * blk_sz + o, _LANES)]
                        buf[pl.ds(o, _LANES)] = plsc.load_gather(
                            row_v, [ids]
                        )
                    pltpu.async_copy(
                        buf,
                        outT_hbm.at[d, pl.ds(blk * blk_sz, blk_sz)],
                        sem_o,
                    )
                for blk in (n_blocks - 2, n_blocks - 1):
                    pltpu.make_async_copy(
                        out_a if blk % 2 == 0 else out_b,
                        outT_hbm.at[d, pl.ds(blk * blk_sz, blk_sz)],
                        sem_o,
                    ).wait()

    return gather


def kernel(embeddings, indices):
    (B,) = indices.shape
    V, D = embeddings.shape
    gather = _build_gather(D, V, B, embeddings.dtype)
    outT = gather(embeddings.T, indices.astype(jnp.int32))
    return outT.T
